# Initial kernel scaffold; baseline (speedup 1.0000x reference)
import jax
import jax.numpy as jnp
from jax.experimental import pallas as pl


def kernel(x, edge_index, W1, b1, W2, b2):
    raise NotImplementedError("write your pallas kernel here")


# speedup vs baseline: 10.7514x; 10.7514x over previous
import jax
import jax.numpy as jnp
from jax import lax
from jax.experimental import pallas as pl
from jax.experimental.pallas import tpu as pltpu
from jax.experimental.pallas import tpu_sc as plsc

N = 10000
E = 320000
D = 128
HD = 64
NC, NS = 2, 16
RPS = N // NS
CH = 128
NCHUNK = E // CH
BASE_CH = NCHUNK // NS
EXTRA_CH = NCHUNK - BASE_CH * NS
EPS = E // (NC * NS)
STG = 2000
NPAD = 10240

_MESH = plsc.VectorSubcoreMesh(
    core_axis_name="c", subcore_axis_name="s", num_cores=NC, num_subcores=NS)
_SC_PARAMS = pltpu.CompilerParams(use_tc_tiling_on_sc=False)


def _mp_body(g_ref, srcs_ref, dst_ref, out_ref, idx_src, idx_dst, rows, acc, sem):
    c = lax.axis_index("c")
    s = lax.axis_index("s")
    r0 = s * RPS
    pltpu.sync_copy(g_ref.at[pl.ds(c * N + r0, RPS)], acc.at[pl.ds(r0, RPS)])
    plsc.subcore_barrier()
    start = BASE_CH * s + jnp.minimum(s, EXTRA_CH)
    nch = jnp.where(s < EXTRA_CH, BASE_CH + 1, BASE_CH)

    def chunk(j, carry):
        e0 = (start + j) * CH
        pltpu.sync_copy(srcs_ref.at[c, pl.ds(e0, CH)], idx_src)
        pltpu.sync_copy(dst_ref.at[pl.ds(e0, CH)], idx_dst)
        pltpu.async_copy(g_ref.at[idx_src], rows, sem).wait()
        pltpu.sync_copy(rows, acc.at[idx_dst], add=True)
        return carry

    lax.fori_loop(0, nch, chunk, 0)
    plsc.subcore_barrier()
    pltpu.sync_copy(acc.at[pl.ds(r0, RPS)], out_ref.at[pl.ds(c * N + r0, RPS)])


_mp_call = pl.kernel(
    _mp_body,
    out_type=jax.ShapeDtypeStruct((2 * N, HD), jnp.float32),
    mesh=_MESH,
    scratch_types=[
        pltpu.VMEM((CH,), jnp.int32),
        pltpu.VMEM((CH,), jnp.int32),
        pltpu.VMEM((CH, HD), jnp.float32),
        pltpu.VMEM_SHARED((N, HD), jnp.float32),
        pltpu.SemaphoreType.DMA,
    ],
    compiler_params=_SC_PARAMS,
)


DCH = 128
DNCH = E // (NC * NS * DCH)
DEG_NCHUNK = E // DCH
DEG_BASE = DEG_NCHUNK // (NC * NS)
DEG_EXTRA = DEG_NCHUNK - DEG_BASE * NC * NS


def _deg_body(dst_ref, degp_ref, idx_dst, ones_rows, zbuf, acc):
    c = lax.axis_index("c")
    s = lax.axis_index("s")
    w = c * NS + s
    ids16 = lax.iota(jnp.int32, 16)
    e0vec = jnp.where(ids16 == 0, 1.0, 0.0).astype(jnp.float32)
    zeros16 = jnp.zeros((16,), jnp.float32)
    nrow = NPAD // NS

    def fill(i, carry):
        ones_rows[i] = e0vec
        zbuf[i] = zeros16
        return carry

    lax.fori_loop(0, nrow, fill, 0)
    pltpu.sync_copy(zbuf, acc.at[pl.ds(s * nrow, nrow)])
    plsc.subcore_barrier()

    start = DEG_BASE * w + jnp.minimum(w, DEG_EXTRA)
    nch = jnp.where(w < DEG_EXTRA, DEG_BASE + 1, DEG_BASE)

    def chunk(j, carry):
        e0 = (start + j) * DCH
        pltpu.sync_copy(dst_ref.at[pl.ds(e0, DCH)], idx_dst)
        pltpu.sync_copy(ones_rows.at[pl.ds(0, DCH)], acc.at[idx_dst], add=True)
        return carry

    lax.fori_loop(0, nch, chunk, 0)
    plsc.subcore_barrier()
    pltpu.sync_copy(acc.at[pl.ds(s * nrow, nrow)],
                    degp_ref.at[c, pl.ds(s * nrow, nrow)])


_deg_call = pl.kernel(
    _deg_body,
    out_type=jax.ShapeDtypeStruct((NC, NPAD, 16), jnp.float32),
    mesh=_MESH,
    scratch_types=[
        pltpu.VMEM((DCH,), jnp.int32),
        pltpu.VMEM((NPAD // NS, 16), jnp.float32),
        pltpu.VMEM((NPAD // NS, 16), jnp.float32),
        pltpu.VMEM_SHARED((NPAD, 16), jnp.float32),
    ],
    compiler_params=_SC_PARAMS,
)

_R = 1000
_NR = N // _R
_PREC = lax.Precision.HIGHEST


def _mm1_body(x_ref, w_ref, deg_ref, o_ref):
    dinv = lax.rsqrt(deg_ref[...])
    o_ref[...] = jnp.dot(x_ref[...], w_ref[0], precision=_PREC,
                         preferred_element_type=jnp.float32) * dinv


def _mid_body(m0_ref, m1_ref, deg_ref, b1_ref, w_ref, o_ref):
    dinv = lax.rsqrt(deg_ref[...])
    h = jnp.concatenate([m0_ref[...], m1_ref[...]], axis=1)
    x2 = jnp.maximum(h * dinv + b1_ref[...], 0.0)
    o_ref[...] = jnp.dot(x2, w_ref[0], precision=_PREC,
                         preferred_element_type=jnp.float32) * dinv


def _fin_body(m0_ref, m1_ref, deg_ref, b2_ref, o_ref):
    dinv = lax.rsqrt(deg_ref[...])
    h = jnp.concatenate([m0_ref[...], m1_ref[...]], axis=1)
    o_ref[...] = h * dinv + b2_ref[...]


def _mm1(x, W1, deg2d):
    return pl.pallas_call(
        _mm1_body,
        grid=(NC, _NR),
        in_specs=[
            pl.BlockSpec((_R, D), lambda c, i: (i, 0)),
            pl.BlockSpec((1, D, HD), lambda c, i: (c, 0, 0)),
            pl.BlockSpec((_R, 1), lambda c, i: (i, 0)),
        ],
        out_specs=pl.BlockSpec((_R, HD), lambda c, i: (c * _NR + i, 0)),
        out_shape=jax.ShapeDtypeStruct((2 * N, HD), jnp.float32),
    )(x, W1, deg2d)


def _mid(m, deg2d, b1r, W2):
    return pl.pallas_call(
        _mid_body,
        grid=(NC, _NR),
        in_specs=[
            pl.BlockSpec((_R, HD), lambda c, i: (i, 0)),
            pl.BlockSpec((_R, HD), lambda c, i: (_NR + i, 0)),
            pl.BlockSpec((_R, 1), lambda c, i: (i, 0)),
            pl.BlockSpec((1, D), lambda c, i: (0, 0)),
            pl.BlockSpec((1, D, HD), lambda c, i: (c, 0, 0)),
        ],
        out_specs=pl.BlockSpec((_R, HD), lambda c, i: (c * _NR + i, 0)),
        out_shape=jax.ShapeDtypeStruct((2 * N, HD), jnp.float32),
    )(m, m, deg2d, b1r, W2)


def _fin(m, deg2d, b2r):
    return pl.pallas_call(
        _fin_body,
        grid=(_NR,),
        in_specs=[
            pl.BlockSpec((_R, HD), lambda i: (i, 0)),
            pl.BlockSpec((_R, HD), lambda i: (_NR + i, 0)),
            pl.BlockSpec((_R, 1), lambda i: (i, 0)),
            pl.BlockSpec((1, D), lambda i: (0, 0)),
        ],
        out_specs=pl.BlockSpec((_R, D), lambda i: (i, 0)),
        out_shape=jax.ShapeDtypeStruct((N, D), jnp.float32),
    )(m, m, deg2d, b2r)


def kernel(x, edge_index, W1, b1, W2, b2):
    src = edge_index[0]
    dst = edge_index[1]
    srcs = jnp.stack([src, src + N])
    degp = _deg_call(dst)[:, :N, 0]
    deg2d = (degp[0] + degp[1] + 1.0)[:, None]
    W1s = jnp.stack([W1[:, :HD], W1[:, HD:]])
    W2s = jnp.stack([W2[:, :HD], W2[:, HD:]])
    g1 = _mm1(x, W1s, deg2d)
    m1 = _mp_call(g1, srcs, dst)
    g2 = _mid(m1, deg2d, b1[None], W2s)
    m2 = _mp_call(g2, srcs, dst)
    return _fin(m2, deg2d, b2[None])


# speedup vs baseline: 12.3060x; 1.1446x over previous
import jax
import jax.numpy as jnp
from jax import lax
from jax.experimental import pallas as pl
from jax.experimental.pallas import tpu as pltpu
from jax.experimental.pallas import tpu_sc as plsc

N = 10000
E = 320000
D = 128
HD = 64
NC, NS = 2, 16
RPS = N // NS
CH = 128
NCHUNK = E // CH
BASE_CH = NCHUNK // NS
EXTRA_CH = NCHUNK - BASE_CH * NS
EPS = E // (NC * NS)
STG = 2000
NPAD = 10240

_MESH = plsc.VectorSubcoreMesh(
    core_axis_name="c", subcore_axis_name="s", num_cores=NC, num_subcores=NS)
_SC_PARAMS = pltpu.CompilerParams(use_tc_tiling_on_sc=False)


EPAD = 327680
NCHS = EPAD // (NS * CH)
AROW = N + 48


def _mp_body(g_ref, srcs_ref, dstp_ref, out_ref, sidx, didx, rows0, rows1,
             acc, gsem0, gsem1, isem):
    c = lax.axis_index("c")
    s = lax.axis_index("s")
    r0 = s * RPS
    pltpu.async_copy(srcs_ref.at[c, pl.ds(s * NCHS, NCHS)], sidx, isem)
    pltpu.async_copy(dstp_ref.at[pl.ds(s * NCHS, NCHS)], didx, isem)
    pltpu.sync_copy(g_ref.at[pl.ds(c * N + r0, RPS)], acc.at[pl.ds(r0, RPS)])
    pltpu.make_async_copy(srcs_ref.at[c, pl.ds(s * NCHS, NCHS)], sidx, isem).wait()
    pltpu.make_async_copy(dstp_ref.at[pl.ds(s * NCHS, NCHS)], didx, isem).wait()
    plsc.subcore_barrier()

    pltpu.async_copy(g_ref.at[sidx.at[0]], rows0, gsem0)

    def blk(b, carry):
        j0 = 2 * b
        pltpu.async_copy(g_ref.at[sidx.at[j0 + 1]], rows1, gsem1)
        pltpu.make_async_copy(g_ref.at[sidx.at[j0]], rows0, gsem0).wait()
        pltpu.sync_copy(rows0, acc.at[didx.at[j0]], add=True)

        @pl.when(b + 1 < NCHS // 2)
        def _():
            pltpu.async_copy(g_ref.at[sidx.at[j0 + 2]], rows0, gsem0)

        pltpu.make_async_copy(g_ref.at[sidx.at[j0 + 1]], rows1, gsem1).wait()
        pltpu.sync_copy(rows1, acc.at[didx.at[j0 + 1]], add=True)
        return carry

    lax.fori_loop(0, NCHS // 2, blk, 0)
    plsc.subcore_barrier()
    pltpu.sync_copy(acc.at[pl.ds(r0, RPS)], out_ref.at[pl.ds(c * N + r0, RPS)])


_mp_call = pl.kernel(
    _mp_body,
    out_type=jax.ShapeDtypeStruct((2 * N, HD), jnp.float32),
    mesh=_MESH,
    scratch_types=[
        pltpu.VMEM((NCHS, CH), jnp.int32),
        pltpu.VMEM((NCHS, CH), jnp.int32),
        pltpu.VMEM((CH, HD), jnp.float32),
        pltpu.VMEM((CH, HD), jnp.float32),
        pltpu.VMEM_SHARED((AROW, HD), jnp.float32),
        pltpu.SemaphoreType.DMA,
        pltpu.SemaphoreType.DMA,
        pltpu.SemaphoreType.DMA,
    ],
    compiler_params=_SC_PARAMS,
)


DCH = 128
DNCH = E // (NC * NS * DCH)
DEG_NCHUNK = E // DCH
DEG_BASE = DEG_NCHUNK // (NC * NS)
DEG_EXTRA = DEG_NCHUNK - DEG_BASE * NC * NS


def _deg_body(dst_ref, degp_ref, idx_dst, ones_rows, zbuf, acc):
    c = lax.axis_index("c")
    s = lax.axis_index("s")
    w = c * NS + s
    ids16 = lax.iota(jnp.int32, 16)
    e0vec = jnp.where(ids16 == 0, 1.0, 0.0).astype(jnp.float32)
    zeros16 = jnp.zeros((16,), jnp.float32)
    nrow = NPAD // NS

    def fill(i, carry):
        ones_rows[i] = e0vec
        zbuf[i] = zeros16
        return carry

    lax.fori_loop(0, nrow, fill, 0)
    pltpu.sync_copy(zbuf, acc.at[pl.ds(s * nrow, nrow)])
    plsc.subcore_barrier()

    start = DEG_BASE * w + jnp.minimum(w, DEG_EXTRA)
    nch = jnp.where(w < DEG_EXTRA, DEG_BASE + 1, DEG_BASE)

    def chunk(j, carry):
        e0 = (start + j) * DCH
        pltpu.sync_copy(dst_ref.at[pl.ds(e0, DCH)], idx_dst)
        pltpu.sync_copy(ones_rows.at[pl.ds(0, DCH)], acc.at[idx_dst], add=True)
        return carry

    lax.fori_loop(0, nch, chunk, 0)
    plsc.subcore_barrier()
    pltpu.sync_copy(acc.at[pl.ds(s * nrow, nrow)],
                    degp_ref.at[c, pl.ds(s * nrow, nrow)])


_deg_call = pl.kernel(
    _deg_body,
    out_type=jax.ShapeDtypeStruct((NC, NPAD, 16), jnp.float32),
    mesh=_MESH,
    scratch_types=[
        pltpu.VMEM((DCH,), jnp.int32),
        pltpu.VMEM((NPAD // NS, 16), jnp.float32),
        pltpu.VMEM((NPAD // NS, 16), jnp.float32),
        pltpu.VMEM_SHARED((NPAD, 16), jnp.float32),
    ],
    compiler_params=_SC_PARAMS,
)

_R = 1000
_NR = N // _R
_PREC = lax.Precision.HIGHEST


def _mm1_body(x_ref, w_ref, deg_ref, o_ref):
    dinv = lax.rsqrt(deg_ref[...])
    o_ref[...] = jnp.dot(x_ref[...], w_ref[0], precision=_PREC,
                         preferred_element_type=jnp.float32) * dinv


def _mid_body(m0_ref, m1_ref, deg_ref, b1_ref, w_ref, o_ref):
    dinv = lax.rsqrt(deg_ref[...])
    h = jnp.concatenate([m0_ref[...], m1_ref[...]], axis=1)
    x2 = jnp.maximum(h * dinv + b1_ref[...], 0.0)
    o_ref[...] = jnp.dot(x2, w_ref[0], precision=_PREC,
                         preferred_element_type=jnp.float32) * dinv


def _fin_body(m0_ref, m1_ref, deg_ref, b2_ref, o_ref):
    dinv = lax.rsqrt(deg_ref[...])
    h = jnp.concatenate([m0_ref[...], m1_ref[...]], axis=1)
    o_ref[...] = h * dinv + b2_ref[...]


def _mm1(x, W1, deg2d):
    return pl.pallas_call(
        _mm1_body,
        grid=(NC, _NR),
        in_specs=[
            pl.BlockSpec((_R, D), lambda c, i: (i, 0)),
            pl.BlockSpec((1, D, HD), lambda c, i: (c, 0, 0)),
            pl.BlockSpec((_R, 1), lambda c, i: (i, 0)),
        ],
        out_specs=pl.BlockSpec((_R, HD), lambda c, i: (c * _NR + i, 0)),
        out_shape=jax.ShapeDtypeStruct((2 * N, HD), jnp.float32),
    )(x, W1, deg2d)


def _mid(m, deg2d, b1r, W2):
    return pl.pallas_call(
        _mid_body,
        grid=(NC, _NR),
        in_specs=[
            pl.BlockSpec((_R, HD), lambda c, i: (i, 0)),
            pl.BlockSpec((_R, HD), lambda c, i: (_NR + i, 0)),
            pl.BlockSpec((_R, 1), lambda c, i: (i, 0)),
            pl.BlockSpec((1, D), lambda c, i: (0, 0)),
            pl.BlockSpec((1, D, HD), lambda c, i: (c, 0, 0)),
        ],
        out_specs=pl.BlockSpec((_R, HD), lambda c, i: (c * _NR + i, 0)),
        out_shape=jax.ShapeDtypeStruct((2 * N, HD), jnp.float32),
    )(m, m, deg2d, b1r, W2)


def _fin(m, deg2d, b2r):
    return pl.pallas_call(
        _fin_body,
        grid=(_NR,),
        in_specs=[
            pl.BlockSpec((_R, HD), lambda i: (i, 0)),
            pl.BlockSpec((_R, HD), lambda i: (_NR + i, 0)),
            pl.BlockSpec((_R, 1), lambda i: (i, 0)),
            pl.BlockSpec((1, D), lambda i: (0, 0)),
        ],
        out_specs=pl.BlockSpec((_R, D), lambda i: (i, 0)),
        out_shape=jax.ShapeDtypeStruct((N, D), jnp.float32),
    )(m, m, deg2d, b2r)


def kernel(x, edge_index, W1, b1, W2, b2):
    src = edge_index[0]
    dst = edge_index[1]
    npad = EPAD - E
    srcp = jnp.concatenate([src, jnp.zeros((npad,), jnp.int32)])
    srcs = jnp.stack([srcp, srcp + N]).reshape(NC, EPAD // CH, CH)
    dstp = jnp.concatenate([dst, jnp.full((npad,), N, jnp.int32)])
    dstp = dstp.reshape(EPAD // CH, CH)
    degp = _deg_call(dst)[:, :N, 0]
    deg2d = (degp[0] + degp[1] + 1.0)[:, None]
    W1s = jnp.stack([W1[:, :HD], W1[:, HD:]])
    W2s = jnp.stack([W2[:, :HD], W2[:, HD:]])
    g1 = _mm1(x, W1s, deg2d)
    m1 = _mp_call(g1, srcs, dstp)
    g2 = _mid(m1, deg2d, b1[None], W2s)
    m2 = _mp_call(g2, srcs, dstp)
    return _fin(m2, deg2d, b2[None])


# speedup vs baseline: 12.5844x; 1.0226x over previous
import jax
import jax.numpy as jnp
from jax import lax
from jax.experimental import pallas as pl
from jax.experimental.pallas import tpu as pltpu
from jax.experimental.pallas import tpu_sc as plsc

N = 10000
E = 320000
D = 128
HD = 64
NC, NS = 2, 16
RPS = N // NS
CH = 128
NCHUNK = E // CH
BASE_CH = NCHUNK // NS
EXTRA_CH = NCHUNK - BASE_CH * NS
EPS = E // (NC * NS)
STG = 2000
NPAD = 10240

_MESH = plsc.VectorSubcoreMesh(
    core_axis_name="c", subcore_axis_name="s", num_cores=NC, num_subcores=NS)
_SC_PARAMS = pltpu.CompilerParams(use_tc_tiling_on_sc=False)


EPAD = 327680
NCHS = EPAD // (NS * CH)
AROW = N + 48


EPAD = 327680
NCHS = EPAD // (NS * CH)
NBLK = NCHS // 4
AROW = N + 48


def _mp_body(g_ref, srcs_ref, dstp_ref, out_ref, sidx, didx,
             rows0, rows1, rows2, rows3, acc,
             g0, g1, g2, g3, s0, s1, s2, s3, isem):
    rows = [rows0, rows1, rows2, rows3]
    gsem = [g0, g1, g2, g3]
    ssem = [s0, s1, s2, s3]
    c = lax.axis_index("c")
    s = lax.axis_index("s")
    r0 = s * RPS
    pltpu.async_copy(srcs_ref.at[c, pl.ds(s * NCHS, NCHS)], sidx, isem)
    pltpu.async_copy(dstp_ref.at[pl.ds(s * NCHS, NCHS)], didx, isem)
    pltpu.sync_copy(g_ref.at[pl.ds(c * N + r0, RPS)], acc.at[pl.ds(r0, RPS)])
    pltpu.make_async_copy(srcs_ref.at[c, pl.ds(s * NCHS, NCHS)], sidx, isem).wait()
    pltpu.make_async_copy(dstp_ref.at[pl.ds(s * NCHS, NCHS)], didx, isem).wait()
    plsc.subcore_barrier()

    def g_start(j, b):
        pltpu.async_copy(g_ref.at[sidx.at[j]], rows[b], gsem[b])

    def g_wait(j, b):
        pltpu.make_async_copy(g_ref.at[sidx.at[j]], rows[b], gsem[b]).wait()

    def s_start(j, b):
        pltpu.async_copy(rows[b], acc.at[didx.at[j]], ssem[b], add=True)

    def s_wait(j, b):
        pltpu.make_async_copy(rows[b], acc.at[didx.at[j]], ssem[b]).wait()

    g_start(0, 0)
    g_start(1, 1)

    def blk(k, carry):
        for b in range(4):
            j = 4 * k + b
            nb = (b + 2) % 4
            g_wait(j, b)
            s_start(j, b)
            if b < 2:
                @pl.when(k > 0)
                def _():
                    s_wait(j - 2, nb)
                g_start(j + 2, nb)
            else:
                s_wait(j - 2, nb)

                @pl.when(k + 1 < NBLK)
                def _():
                    g_start(j + 2, nb)
        return carry

    lax.fori_loop(0, NBLK, blk, 0)
    s_wait(NCHS - 2, 2)
    s_wait(NCHS - 1, 3)
    plsc.subcore_barrier()
    pltpu.sync_copy(acc.at[pl.ds(r0, RPS)], out_ref.at[pl.ds(c * N + r0, RPS)])


_mp_call = pl.kernel(
    _mp_body,
    out_type=jax.ShapeDtypeStruct((2 * N, HD), jnp.float32),
    mesh=_MESH,
    scratch_types=[
        pltpu.VMEM((NCHS, CH), jnp.int32),
        pltpu.VMEM((NCHS, CH), jnp.int32),
        pltpu.VMEM((CH, HD), jnp.float32),
        pltpu.VMEM((CH, HD), jnp.float32),
        pltpu.VMEM((CH, HD), jnp.float32),
        pltpu.VMEM((CH, HD), jnp.float32),
        pltpu.VMEM_SHARED((AROW, HD), jnp.float32),
        pltpu.SemaphoreType.DMA,
        pltpu.SemaphoreType.DMA,
        pltpu.SemaphoreType.DMA,
        pltpu.SemaphoreType.DMA,
        pltpu.SemaphoreType.DMA,
        pltpu.SemaphoreType.DMA,
        pltpu.SemaphoreType.DMA,
        pltpu.SemaphoreType.DMA,
        pltpu.SemaphoreType.DMA,
    ],
    compiler_params=_SC_PARAMS,
)


DCH = 128
DNCH = E // (NC * NS * DCH)
DEG_NCHUNK = E // DCH
DEG_BASE = DEG_NCHUNK // (NC * NS)
DEG_EXTRA = DEG_NCHUNK - DEG_BASE * NC * NS


def _deg_body(dst_ref, degp_ref, idx_dst, ones_rows, zbuf, acc):
    c = lax.axis_index("c")
    s = lax.axis_index("s")
    w = c * NS + s
    ids16 = lax.iota(jnp.int32, 16)
    e0vec = jnp.where(ids16 == 0, 1.0, 0.0).astype(jnp.float32)
    zeros16 = jnp.zeros((16,), jnp.float32)
    nrow = NPAD // NS

    def fill(i, carry):
        ones_rows[i] = e0vec
        zbuf[i] = zeros16
        return carry

    lax.fori_loop(0, nrow, fill, 0)
    pltpu.sync_copy(zbuf, acc.at[pl.ds(s * nrow, nrow)])
    plsc.subcore_barrier()

    start = DEG_BASE * w + jnp.minimum(w, DEG_EXTRA)
    nch = jnp.where(w < DEG_EXTRA, DEG_BASE + 1, DEG_BASE)

    def chunk(j, carry):
        e0 = (start + j) * DCH
        pltpu.sync_copy(dst_ref.at[pl.ds(e0, DCH)], idx_dst)
        pltpu.sync_copy(ones_rows.at[pl.ds(0, DCH)], acc.at[idx_dst], add=True)
        return carry

    lax.fori_loop(0, nch, chunk, 0)
    plsc.subcore_barrier()
    pltpu.sync_copy(acc.at[pl.ds(s * nrow, nrow)],
                    degp_ref.at[c, pl.ds(s * nrow, nrow)])


_deg_call = pl.kernel(
    _deg_body,
    out_type=jax.ShapeDtypeStruct((NC, NPAD, 16), jnp.float32),
    mesh=_MESH,
    scratch_types=[
        pltpu.VMEM((DCH,), jnp.int32),
        pltpu.VMEM((NPAD // NS, 16), jnp.float32),
        pltpu.VMEM((NPAD // NS, 16), jnp.float32),
        pltpu.VMEM_SHARED((NPAD, 16), jnp.float32),
    ],
    compiler_params=_SC_PARAMS,
)

_R = 1000
_NR = N // _R
_PREC = lax.Precision.HIGHEST


def _mm1_body(x_ref, w_ref, deg_ref, o_ref):
    dinv = lax.rsqrt(deg_ref[...])
    o_ref[...] = jnp.dot(x_ref[...], w_ref[0], precision=_PREC,
                         preferred_element_type=jnp.float32) * dinv


def _mid_body(m0_ref, m1_ref, deg_ref, b1_ref, w_ref, o_ref):
    dinv = lax.rsqrt(deg_ref[...])
    h = jnp.concatenate([m0_ref[...], m1_ref[...]], axis=1)
    x2 = jnp.maximum(h * dinv + b1_ref[...], 0.0)
    o_ref[...] = jnp.dot(x2, w_ref[0], precision=_PREC,
                         preferred_element_type=jnp.float32) * dinv


def _fin_body(m0_ref, m1_ref, deg_ref, b2_ref, o_ref):
    dinv = lax.rsqrt(deg_ref[...])
    h = jnp.concatenate([m0_ref[...], m1_ref[...]], axis=1)
    o_ref[...] = h * dinv + b2_ref[...]


def _mm1(x, W1, deg2d):
    return pl.pallas_call(
        _mm1_body,
        grid=(NC, _NR),
        in_specs=[
            pl.BlockSpec((_R, D), lambda c, i: (i, 0)),
            pl.BlockSpec((1, D, HD), lambda c, i: (c, 0, 0)),
            pl.BlockSpec((_R, 1), lambda c, i: (i, 0)),
        ],
        out_specs=pl.BlockSpec((_R, HD), lambda c, i: (c * _NR + i, 0)),
        out_shape=jax.ShapeDtypeStruct((2 * N, HD), jnp.float32),
    )(x, W1, deg2d)


def _mid(m, deg2d, b1r, W2):
    return pl.pallas_call(
        _mid_body,
        grid=(NC, _NR),
        in_specs=[
            pl.BlockSpec((_R, HD), lambda c, i: (i, 0)),
            pl.BlockSpec((_R, HD), lambda c, i: (_NR + i, 0)),
            pl.BlockSpec((_R, 1), lambda c, i: (i, 0)),
            pl.BlockSpec((1, D), lambda c, i: (0, 0)),
            pl.BlockSpec((1, D, HD), lambda c, i: (c, 0, 0)),
        ],
        out_specs=pl.BlockSpec((_R, HD), lambda c, i: (c * _NR + i, 0)),
        out_shape=jax.ShapeDtypeStruct((2 * N, HD), jnp.float32),
    )(m, m, deg2d, b1r, W2)


def _fin(m, deg2d, b2r):
    return pl.pallas_call(
        _fin_body,
        grid=(_NR,),
        in_specs=[
            pl.BlockSpec((_R, HD), lambda i: (i, 0)),
            pl.BlockSpec((_R, HD), lambda i: (_NR + i, 0)),
            pl.BlockSpec((_R, 1), lambda i: (i, 0)),
            pl.BlockSpec((1, D), lambda i: (0, 0)),
        ],
        out_specs=pl.BlockSpec((_R, D), lambda i: (i, 0)),
        out_shape=jax.ShapeDtypeStruct((N, D), jnp.float32),
    )(m, m, deg2d, b2r)


def kernel(x, edge_index, W1, b1, W2, b2):
    src = edge_index[0]
    dst = edge_index[1]
    npad = EPAD - E
    srcp = jnp.concatenate([src, jnp.zeros((npad,), jnp.int32)])
    srcs = jnp.stack([srcp, srcp + N]).reshape(NC, EPAD // CH, CH)
    dstp = jnp.concatenate([dst, jnp.full((npad,), N, jnp.int32)])
    dstp = dstp.reshape(EPAD // CH, CH)
    degp = _deg_call(dst)[:, :N, 0]
    deg2d = (degp[0] + degp[1] + 1.0)[:, None]
    W1s = jnp.stack([W1[:, :HD], W1[:, HD:]])
    W2s = jnp.stack([W2[:, :HD], W2[:, HD:]])
    g1 = _mm1(x, W1s, deg2d)
    m1 = _mp_call(g1, srcs, dstp)
    g2 = _mid(m1, deg2d, b1[None], W2s)
    m2 = _mp_call(g2, srcs, dstp)
    return _fin(m2, deg2d, b2[None])


# speedup vs baseline: 12.7273x; 1.0114x over previous
import jax
import jax.numpy as jnp
from jax import lax
from jax.experimental import pallas as pl
from jax.experimental.pallas import tpu as pltpu
from jax.experimental.pallas import tpu_sc as plsc

N = 10000
E = 320000
D = 128
HD = 64
NC, NS = 2, 16
RPS = N // NS
CH = 128
NCHUNK = E // CH
BASE_CH = NCHUNK // NS
EXTRA_CH = NCHUNK - BASE_CH * NS
EPS = E // (NC * NS)
STG = 2000
NPAD = 10240

_MESH = plsc.VectorSubcoreMesh(
    core_axis_name="c", subcore_axis_name="s", num_cores=NC, num_subcores=NS)
_SC_PARAMS = pltpu.CompilerParams(use_tc_tiling_on_sc=False)


EPAD = 327680
NCHS = EPAD // (NS * CH)
AROW = N + 48


EPAD = 327680
NCHS = EPAD // (NS * CH)
NBLK = NCHS // 4
AROW = N + 48


def _mp_body(g_ref, srcs_ref, dstp_ref, out_ref, sidx, didx,
             rows0, rows1, rows2, rows3, acc,
             g0, g1, g2, g3, s0, s1, s2, s3, isem):
    rows = [rows0, rows1, rows2, rows3]
    gsem = [g0, g1, g2, g3]
    ssem = [s0, s1, s2, s3]
    c = lax.axis_index("c")
    s = lax.axis_index("s")
    r0 = s * RPS
    pltpu.async_copy(srcs_ref.at[c, pl.ds(s * NCHS, NCHS)], sidx, isem)
    pltpu.async_copy(dstp_ref.at[pl.ds(s * NCHS, NCHS)], didx, isem)
    pltpu.sync_copy(g_ref.at[pl.ds(c * N + r0, RPS)], acc.at[pl.ds(r0, RPS)])
    pltpu.make_async_copy(srcs_ref.at[c, pl.ds(s * NCHS, NCHS)], sidx, isem).wait()
    pltpu.make_async_copy(dstp_ref.at[pl.ds(s * NCHS, NCHS)], didx, isem).wait()
    plsc.subcore_barrier()

    def g_start(j, b):
        pltpu.async_copy(g_ref.at[sidx.at[j]], rows[b], gsem[b])

    def g_wait(j, b):
        pltpu.make_async_copy(g_ref.at[sidx.at[j]], rows[b], gsem[b]).wait()

    def s_start(j, b):
        pltpu.async_copy(rows[b], acc.at[didx.at[j]], ssem[b], add=True)

    def s_wait(j, b):
        pltpu.make_async_copy(rows[b], acc.at[didx.at[j]], ssem[b]).wait()

    g_start(0, 0)
    g_start(1, 1)

    def blk(k, carry):
        for b in range(4):
            j = 4 * k + b
            nb = (b + 2) % 4
            g_wait(j, b)
            s_start(j, b)
            if b < 2:
                @pl.when(k > 0)
                def _():
                    s_wait(j - 2, nb)
                g_start(j + 2, nb)
            else:
                s_wait(j - 2, nb)

                @pl.when(k + 1 < NBLK)
                def _():
                    g_start(j + 2, nb)
        return carry

    lax.fori_loop(0, NBLK, blk, 0)
    s_wait(NCHS - 2, 2)
    s_wait(NCHS - 1, 3)
    plsc.subcore_barrier()
    pltpu.sync_copy(acc.at[pl.ds(r0, RPS)], out_ref.at[pl.ds(c * N + r0, RPS)])


_mp_call = pl.kernel(
    _mp_body,
    out_type=jax.ShapeDtypeStruct((2 * N, HD), jnp.float32),
    mesh=_MESH,
    scratch_types=[
        pltpu.VMEM((NCHS, CH), jnp.int32),
        pltpu.VMEM((NCHS, CH), jnp.int32),
        pltpu.VMEM((CH, HD), jnp.float32),
        pltpu.VMEM((CH, HD), jnp.float32),
        pltpu.VMEM((CH, HD), jnp.float32),
        pltpu.VMEM((CH, HD), jnp.float32),
        pltpu.VMEM_SHARED((AROW, HD), jnp.float32),
        pltpu.SemaphoreType.DMA,
        pltpu.SemaphoreType.DMA,
        pltpu.SemaphoreType.DMA,
        pltpu.SemaphoreType.DMA,
        pltpu.SemaphoreType.DMA,
        pltpu.SemaphoreType.DMA,
        pltpu.SemaphoreType.DMA,
        pltpu.SemaphoreType.DMA,
        pltpu.SemaphoreType.DMA,
    ],
    compiler_params=_SC_PARAMS,
)


DEG_NCHS = EPAD // (NC * NS * CH)
DEG_NBLK = DEG_NCHS // 4


def _deg_body(dstp_ref, degp_ref, didx, ones_rows, zbuf, acc,
              s0, s1, s2, s3, isem):
    ssem = [s0, s1, s2, s3]
    c = lax.axis_index("c")
    s = lax.axis_index("s")
    w = c * NS + s
    pltpu.async_copy(dstp_ref.at[pl.ds(w * DEG_NCHS, DEG_NCHS)], didx, isem)
    ids16 = lax.iota(jnp.int32, 16)
    e0vec = jnp.where(ids16 == 0, 1.0, 0.0).astype(jnp.float32)
    zeros16 = jnp.zeros((16,), jnp.float32)
    nrow = NPAD // NS

    def fill(i, carry):
        zbuf[i] = zeros16

        @pl.when(i < CH)
        def _():
            ones_rows[i] = e0vec

        return carry

    lax.fori_loop(0, nrow, fill, 0)
    pltpu.sync_copy(zbuf, acc.at[pl.ds(s * nrow, nrow)])
    pltpu.make_async_copy(dstp_ref.at[pl.ds(w * DEG_NCHS, DEG_NCHS)], didx,
                          isem).wait()
    plsc.subcore_barrier()

    def s_start(j, b):
        pltpu.async_copy(ones_rows, acc.at[didx.at[j]], ssem[b], add=True)

    def s_wait(j, b):
        pltpu.make_async_copy(ones_rows, acc.at[didx.at[j]], ssem[b]).wait()

    def blk(k, carry):
        for b in range(4):
            j = 4 * k + b

            @pl.when(k > 0)
            def _():
                s_wait(j - 4, b)

            s_start(j, b)
        return carry

    lax.fori_loop(0, DEG_NBLK, blk, 0)
    for b in range(4):
        s_wait(DEG_NCHS - 4 + b, b)
    plsc.subcore_barrier()
    pltpu.sync_copy(acc.at[pl.ds(s * nrow, nrow)],
                    degp_ref.at[c, pl.ds(s * nrow, nrow)])


_deg_call = pl.kernel(
    _deg_body,
    out_type=jax.ShapeDtypeStruct((NC, NPAD, 16), jnp.float32),
    mesh=_MESH,
    scratch_types=[
        pltpu.VMEM((DEG_NCHS, CH), jnp.int32),
        pltpu.VMEM((CH, 16), jnp.float32),
        pltpu.VMEM((NPAD // NS, 16), jnp.float32),
        pltpu.VMEM_SHARED((NPAD, 16), jnp.float32),
        pltpu.SemaphoreType.DMA,
        pltpu.SemaphoreType.DMA,
        pltpu.SemaphoreType.DMA,
        pltpu.SemaphoreType.DMA,
        pltpu.SemaphoreType.DMA,
    ],
    compiler_params=_SC_PARAMS,
)

_R = 1000
_NR = N // _R
_PREC = lax.Precision.HIGHEST


def _mm1_body(x_ref, w_ref, deg_ref, o_ref):
    dinv = lax.rsqrt(deg_ref[...])
    o_ref[...] = jnp.dot(x_ref[...], w_ref[0], precision=_PREC,
                         preferred_element_type=jnp.float32) * dinv


def _mid_body(m0_ref, m1_ref, deg_ref, b1_ref, w_ref, o_ref):
    dinv = lax.rsqrt(deg_ref[...])
    h = jnp.concatenate([m0_ref[...], m1_ref[...]], axis=1)
    x2 = jnp.maximum(h * dinv + b1_ref[...], 0.0)
    o_ref[...] = jnp.dot(x2, w_ref[0], precision=_PREC,
                         preferred_element_type=jnp.float32) * dinv


def _fin_body(m0_ref, m1_ref, deg_ref, b2_ref, o_ref):
    dinv = lax.rsqrt(deg_ref[...])
    h = jnp.concatenate([m0_ref[...], m1_ref[...]], axis=1)
    o_ref[...] = h * dinv + b2_ref[...]


def _mm1(x, W1, deg2d):
    return pl.pallas_call(
        _mm1_body,
        grid=(NC, _NR),
        in_specs=[
            pl.BlockSpec((_R, D), lambda c, i: (i, 0)),
            pl.BlockSpec((1, D, HD), lambda c, i: (c, 0, 0)),
            pl.BlockSpec((_R, 1), lambda c, i: (i, 0)),
        ],
        out_specs=pl.BlockSpec((_R, HD), lambda c, i: (c * _NR + i, 0)),
        out_shape=jax.ShapeDtypeStruct((2 * N, HD), jnp.float32),
    )(x, W1, deg2d)


def _mid(m, deg2d, b1r, W2):
    return pl.pallas_call(
        _mid_body,
        grid=(NC, _NR),
        in_specs=[
            pl.BlockSpec((_R, HD), lambda c, i: (i, 0)),
            pl.BlockSpec((_R, HD), lambda c, i: (_NR + i, 0)),
            pl.BlockSpec((_R, 1), lambda c, i: (i, 0)),
            pl.BlockSpec((1, D), lambda c, i: (0, 0)),
            pl.BlockSpec((1, D, HD), lambda c, i: (c, 0, 0)),
        ],
        out_specs=pl.BlockSpec((_R, HD), lambda c, i: (c * _NR + i, 0)),
        out_shape=jax.ShapeDtypeStruct((2 * N, HD), jnp.float32),
    )(m, m, deg2d, b1r, W2)


def _fin(m, deg2d, b2r):
    return pl.pallas_call(
        _fin_body,
        grid=(_NR,),
        in_specs=[
            pl.BlockSpec((_R, HD), lambda i: (i, 0)),
            pl.BlockSpec((_R, HD), lambda i: (_NR + i, 0)),
            pl.BlockSpec((_R, 1), lambda i: (i, 0)),
            pl.BlockSpec((1, D), lambda i: (0, 0)),
        ],
        out_specs=pl.BlockSpec((_R, D), lambda i: (i, 0)),
        out_shape=jax.ShapeDtypeStruct((N, D), jnp.float32),
    )(m, m, deg2d, b2r)


def kernel(x, edge_index, W1, b1, W2, b2):
    src = edge_index[0]
    dst = edge_index[1]
    npad = EPAD - E
    srcp = jnp.concatenate([src, jnp.zeros((npad,), jnp.int32)])
    srcs = jnp.stack([srcp, srcp + N]).reshape(NC, EPAD // CH, CH)
    dstp = jnp.concatenate([dst, jnp.full((npad,), N, jnp.int32)])
    dstp = dstp.reshape(EPAD // CH, CH)
    degp = _deg_call(dstp)[:, :N, 0]
    deg2d = (degp[0] + degp[1] + 1.0)[:, None]
    W1s = jnp.stack([W1[:, :HD], W1[:, HD:]])
    W2s = jnp.stack([W2[:, :HD], W2[:, HD:]])
    g1 = _mm1(x, W1s, deg2d)
    m1 = _mp_call(g1, srcs, dstp)
    g2 = _mid(m1, deg2d, b1[None], W2s)
    m2 = _mp_call(g2, srcs, dstp)
    return _fin(m2, deg2d, b2[None])


# speedup vs baseline: 12.8811x; 1.0121x over previous
import jax
import jax.numpy as jnp
from jax import lax
from jax.experimental import pallas as pl
from jax.experimental.pallas import tpu as pltpu
from jax.experimental.pallas import tpu_sc as plsc

N = 10000
E = 320000
D = 128
HD = 64
NC, NS = 2, 16
RPS = N // NS
CH = 128
NCHUNK = E // CH
BASE_CH = NCHUNK // NS
EXTRA_CH = NCHUNK - BASE_CH * NS
EPS = E // (NC * NS)
STG = 2000
NPAD = 10240

_MESH = plsc.VectorSubcoreMesh(
    core_axis_name="c", subcore_axis_name="s", num_cores=NC, num_subcores=NS)
_SC_PARAMS = pltpu.CompilerParams(use_tc_tiling_on_sc=False)


EPAD = 327680
NCHS = EPAD // (NS * CH)
AROW = N + 48


EPAD = 327680
NCHS = EPAD // (NS * CH)
NBLK = NCHS // 4
AROW = N + 48


def _mp_body(g_ref, srcs_ref, dstp_ref, out_ref, sidx, didx,
             rows0, rows1, rows2, rows3, acc,
             g0, g1, g2, g3, s0, s1, s2, s3, isem):
    rows = [rows0, rows1, rows2, rows3]
    gsem = [g0, g1, g2, g3]
    ssem = [s0, s1, s2, s3]
    c = lax.axis_index("c")
    s = lax.axis_index("s")
    r0 = s * RPS
    pltpu.async_copy(srcs_ref.at[pl.ds(s * NCHS, NCHS)], sidx, isem)
    pltpu.async_copy(dstp_ref.at[pl.ds(s * NCHS, NCHS)], didx, isem)
    pltpu.sync_copy(g_ref.at[c, pl.ds(r0, RPS)], acc.at[pl.ds(r0, RPS)])
    pltpu.make_async_copy(srcs_ref.at[pl.ds(s * NCHS, NCHS)], sidx, isem).wait()
    pltpu.make_async_copy(dstp_ref.at[pl.ds(s * NCHS, NCHS)], didx, isem).wait()
    plsc.subcore_barrier()

    def g_start(j, b):
        pltpu.async_copy(g_ref.at[c].at[sidx.at[j]], rows[b], gsem[b])

    def g_wait(j, b):
        pltpu.make_async_copy(g_ref.at[c].at[sidx.at[j]], rows[b], gsem[b]).wait()

    def s_start(j, b):
        pltpu.async_copy(rows[b], acc.at[didx.at[j]], ssem[b], add=True)

    def s_wait(j, b):
        pltpu.make_async_copy(rows[b], acc.at[didx.at[j]], ssem[b]).wait()

    g_start(0, 0)
    g_start(1, 1)

    def blk(k, carry):
        for b in range(4):
            j = 4 * k + b
            nb = (b + 2) % 4
            g_wait(j, b)
            s_start(j, b)
            if b < 2:
                @pl.when(k > 0)
                def _():
                    s_wait(j - 2, nb)
                g_start(j + 2, nb)
            else:
                s_wait(j - 2, nb)

                @pl.when(k + 1 < NBLK)
                def _():
                    g_start(j + 2, nb)
        return carry

    lax.fori_loop(0, NBLK, blk, 0)
    s_wait(NCHS - 2, 2)
    s_wait(NCHS - 1, 3)
    plsc.subcore_barrier()
    pltpu.sync_copy(acc.at[pl.ds(r0, RPS)], out_ref.at[c, pl.ds(r0, RPS)])


_mp_call = pl.kernel(
    _mp_body,
    out_type=jax.ShapeDtypeStruct((NC, N, HD), jnp.float32),
    mesh=_MESH,
    scratch_types=[
        pltpu.VMEM((NCHS, CH), jnp.int32),
        pltpu.VMEM((NCHS, CH), jnp.int32),
        pltpu.VMEM((CH, HD), jnp.float32),
        pltpu.VMEM((CH, HD), jnp.float32),
        pltpu.VMEM((CH, HD), jnp.float32),
        pltpu.VMEM((CH, HD), jnp.float32),
        pltpu.VMEM_SHARED((AROW, HD), jnp.float32),
        pltpu.SemaphoreType.DMA,
        pltpu.SemaphoreType.DMA,
        pltpu.SemaphoreType.DMA,
        pltpu.SemaphoreType.DMA,
        pltpu.SemaphoreType.DMA,
        pltpu.SemaphoreType.DMA,
        pltpu.SemaphoreType.DMA,
        pltpu.SemaphoreType.DMA,
        pltpu.SemaphoreType.DMA,
    ],
    compiler_params=_SC_PARAMS,
)


DEG_NCHS = EPAD // (NC * NS * CH)
DEG_NBLK = DEG_NCHS // 4


def _deg_body(dstp_ref, degp_ref, didx, ones_rows, zbuf, acc,
              s0, s1, s2, s3, isem):
    ssem = [s0, s1, s2, s3]
    c = lax.axis_index("c")
    s = lax.axis_index("s")
    w = c * NS + s
    pltpu.async_copy(dstp_ref.at[pl.ds(w * DEG_NCHS, DEG_NCHS)], didx, isem)
    ids16 = lax.iota(jnp.int32, 16)
    e0vec = jnp.where(ids16 == 0, 1.0, 0.0).astype(jnp.float32)
    zeros16 = jnp.zeros((16,), jnp.float32)
    nrow = NPAD // NS

    def fill(i, carry):
        zbuf[i] = zeros16

        @pl.when(i < CH)
        def _():
            ones_rows[i] = e0vec

        return carry

    lax.fori_loop(0, nrow, fill, 0)
    pltpu.sync_copy(zbuf, acc.at[pl.ds(s * nrow, nrow)])
    pltpu.make_async_copy(dstp_ref.at[pl.ds(w * DEG_NCHS, DEG_NCHS)], didx,
                          isem).wait()
    plsc.subcore_barrier()

    def s_start(j, b):
        pltpu.async_copy(ones_rows, acc.at[didx.at[j]], ssem[b], add=True)

    def s_wait(j, b):
        pltpu.make_async_copy(ones_rows, acc.at[didx.at[j]], ssem[b]).wait()

    def blk(k, carry):
        for b in range(4):
            j = 4 * k + b

            @pl.when(k > 0)
            def _():
                s_wait(j - 4, b)

            s_start(j, b)
        return carry

    lax.fori_loop(0, DEG_NBLK, blk, 0)
    for b in range(4):
        s_wait(DEG_NCHS - 4 + b, b)
    plsc.subcore_barrier()
    pltpu.sync_copy(acc.at[pl.ds(s * nrow, nrow)],
                    degp_ref.at[c, pl.ds(s * nrow, nrow)])


_deg_call = pl.kernel(
    _deg_body,
    out_type=jax.ShapeDtypeStruct((NC, NPAD, 16), jnp.float32),
    mesh=_MESH,
    scratch_types=[
        pltpu.VMEM((DEG_NCHS, CH), jnp.int32),
        pltpu.VMEM((CH, 16), jnp.float32),
        pltpu.VMEM((NPAD // NS, 16), jnp.float32),
        pltpu.VMEM_SHARED((NPAD, 16), jnp.float32),
        pltpu.SemaphoreType.DMA,
        pltpu.SemaphoreType.DMA,
        pltpu.SemaphoreType.DMA,
        pltpu.SemaphoreType.DMA,
        pltpu.SemaphoreType.DMA,
    ],
    compiler_params=_SC_PARAMS,
)

_R = 1000
_NR = N // _R
_PREC = lax.Precision.DEFAULT


def _mm1_body(x_ref, w_ref, deg_ref, o_ref):
    dinv = lax.rsqrt(deg_ref[...])
    o_ref[0] = jnp.dot(x_ref[...], w_ref[0], precision=_PREC,
                       preferred_element_type=jnp.float32) * dinv


def _mid_body(m0_ref, m1_ref, deg_ref, b1_ref, w_ref, o_ref):
    dinv = lax.rsqrt(deg_ref[...])
    h = jnp.concatenate([m0_ref[0], m1_ref[0]], axis=1)
    x2 = jnp.maximum(h * dinv + b1_ref[...], 0.0)
    o_ref[0] = jnp.dot(x2, w_ref[0], precision=_PREC,
                       preferred_element_type=jnp.float32) * dinv


def _fin_body(m0_ref, m1_ref, deg_ref, b2_ref, o_ref):
    dinv = lax.rsqrt(deg_ref[...])
    h = jnp.concatenate([m0_ref[0], m1_ref[0]], axis=1)
    o_ref[...] = h * dinv + b2_ref[...]


def _mm1(x, W1, deg2d):
    return pl.pallas_call(
        _mm1_body,
        grid=(NC, _NR),
        in_specs=[
            pl.BlockSpec((_R, D), lambda c, i: (i, 0)),
            pl.BlockSpec((1, D, HD), lambda c, i: (c, 0, 0)),
            pl.BlockSpec((_R, 1), lambda c, i: (i, 0)),
        ],
        out_specs=pl.BlockSpec((1, _R, HD), lambda c, i: (c, i, 0)),
        out_shape=jax.ShapeDtypeStruct((NC, N, HD), jnp.float32),
    )(x, W1, deg2d)


def _mid(m, deg2d, b1r, W2):
    return pl.pallas_call(
        _mid_body,
        grid=(NC, _NR),
        in_specs=[
            pl.BlockSpec((1, _R, HD), lambda c, i: (0, i, 0)),
            pl.BlockSpec((1, _R, HD), lambda c, i: (1, i, 0)),
            pl.BlockSpec((_R, 1), lambda c, i: (i, 0)),
            pl.BlockSpec((1, D), lambda c, i: (0, 0)),
            pl.BlockSpec((1, D, HD), lambda c, i: (c, 0, 0)),
        ],
        out_specs=pl.BlockSpec((1, _R, HD), lambda c, i: (c, i, 0)),
        out_shape=jax.ShapeDtypeStruct((NC, N, HD), jnp.float32),
    )(m, m, deg2d, b1r, W2)


def _fin(m, deg2d, b2r):
    return pl.pallas_call(
        _fin_body,
        grid=(_NR,),
        in_specs=[
            pl.BlockSpec((1, _R, HD), lambda i: (0, i, 0)),
            pl.BlockSpec((1, _R, HD), lambda i: (1, i, 0)),
            pl.BlockSpec((_R, 1), lambda i: (i, 0)),
            pl.BlockSpec((1, D), lambda i: (0, 0)),
        ],
        out_specs=pl.BlockSpec((_R, D), lambda i: (i, 0)),
        out_shape=jax.ShapeDtypeStruct((N, D), jnp.float32),
    )(m, m, deg2d, b2r)


def kernel(x, edge_index, W1, b1, W2, b2):
    src = edge_index[0]
    dst = edge_index[1]
    npad = EPAD - E
    srcs = jnp.concatenate([src, jnp.zeros((npad,), jnp.int32)]).reshape(
        EPAD // CH, CH)
    dstp = jnp.concatenate([dst, jnp.full((npad,), N, jnp.int32)])
    dstp = dstp.reshape(EPAD // CH, CH)
    degp = _deg_call(dstp)[:, :N, 0]
    deg2d = (degp[0] + degp[1] + 1.0)[:, None]
    W1s = jnp.stack([W1[:, :HD], W1[:, HD:]])
    W2s = jnp.stack([W2[:, :HD], W2[:, HD:]])
    g1 = _mm1(x, W1s, deg2d)
    m1 = _mp_call(g1, srcs, dstp)
    g2 = _mid(m1, deg2d, b1[None], W2s)
    m2 = _mp_call(g2, srcs, dstp)
    return _fin(m2, deg2d, b2[None])


# speedup vs baseline: 13.1387x; 1.0200x over previous
import jax
import jax.numpy as jnp
from jax import lax
from jax.experimental import pallas as pl
from jax.experimental.pallas import tpu as pltpu
from jax.experimental.pallas import tpu_sc as plsc

N = 10000
E = 320000
D = 128
HD = 64
NC, NS = 2, 16
RPS = N // NS
CH = 128
NCHUNK = E // CH
BASE_CH = NCHUNK // NS
EXTRA_CH = NCHUNK - BASE_CH * NS
EPS = E // (NC * NS)
STG = 2000
NPAD = 10240

_MESH = plsc.VectorSubcoreMesh(
    core_axis_name="c", subcore_axis_name="s", num_cores=NC, num_subcores=NS)
_SC_PARAMS = pltpu.CompilerParams(use_tc_tiling_on_sc=False)


EPAD = 327680
NCHS = EPAD // (NS * CH)
AROW = N + 48


EPAD = 327680
NCHS = EPAD // (NS * CH)
NBLK = NCHS // 4
AROW = N + 48


def _mp_body(g_ref, srcs_ref, dstp_ref, out_ref, sidx, didx,
             rows0, rows1, rows2, rows3, acc,
             g0, g1, g2, g3, s0, s1, s2, s3, isem):
    rows = [rows0, rows1, rows2, rows3]
    gsem = [g0, g1, g2, g3]
    ssem = [s0, s1, s2, s3]
    c = lax.axis_index("c")
    s = lax.axis_index("s")
    r0 = s * RPS
    pltpu.async_copy(srcs_ref.at[pl.ds(s * NCHS, NCHS)], sidx, isem)
    pltpu.async_copy(dstp_ref.at[pl.ds(s * NCHS, NCHS)], didx, isem)
    pltpu.sync_copy(g_ref.at[c, pl.ds(r0, RPS)], acc.at[pl.ds(r0, RPS)])
    pltpu.make_async_copy(srcs_ref.at[pl.ds(s * NCHS, NCHS)], sidx, isem).wait()
    pltpu.make_async_copy(dstp_ref.at[pl.ds(s * NCHS, NCHS)], didx, isem).wait()
    plsc.subcore_barrier()

    def g_start(j, b):
        pltpu.async_copy(g_ref.at[c].at[sidx.at[j]], rows[b], gsem[b])

    def g_wait(j, b):
        pltpu.make_async_copy(g_ref.at[c].at[sidx.at[j]], rows[b], gsem[b]).wait()

    def s_start(j, b):
        pltpu.async_copy(rows[b], acc.at[didx.at[j]], ssem[b], add=True)

    def s_wait(j, b):
        pltpu.make_async_copy(rows[b], acc.at[didx.at[j]], ssem[b]).wait()

    g_start(0, 0)
    g_start(1, 1)

    def blk(k, carry):
        for b in range(4):
            j = 4 * k + b
            nb = (b + 2) % 4
            g_wait(j, b)
            s_start(j, b)
            if b < 2:
                @pl.when(k > 0)
                def _():
                    s_wait(j - 2, nb)
                g_start(j + 2, nb)
            else:
                s_wait(j - 2, nb)

                @pl.when(k + 1 < NBLK)
                def _():
                    g_start(j + 2, nb)
        return carry

    lax.fori_loop(0, NBLK, blk, 0)
    s_wait(NCHS - 2, 2)
    s_wait(NCHS - 1, 3)
    plsc.subcore_barrier()
    pltpu.sync_copy(acc.at[pl.ds(r0, RPS)], out_ref.at[c, pl.ds(r0, RPS)])


_mp_call = pl.kernel(
    _mp_body,
    out_type=jax.ShapeDtypeStruct((NC, N, HD), jnp.float32),
    mesh=_MESH,
    scratch_types=[
        pltpu.VMEM((NCHS, CH), jnp.int32),
        pltpu.VMEM((NCHS, CH), jnp.int32),
        pltpu.VMEM((CH, HD), jnp.float32),
        pltpu.VMEM((CH, HD), jnp.float32),
        pltpu.VMEM((CH, HD), jnp.float32),
        pltpu.VMEM((CH, HD), jnp.float32),
        pltpu.VMEM_SHARED((AROW, HD), jnp.float32),
        pltpu.SemaphoreType.DMA,
        pltpu.SemaphoreType.DMA,
        pltpu.SemaphoreType.DMA,
        pltpu.SemaphoreType.DMA,
        pltpu.SemaphoreType.DMA,
        pltpu.SemaphoreType.DMA,
        pltpu.SemaphoreType.DMA,
        pltpu.SemaphoreType.DMA,
        pltpu.SemaphoreType.DMA,
    ],
    compiler_params=_SC_PARAMS,
)


DEG_NCHS = EPAD // (NC * NS * CH)
DEG_NBLK = DEG_NCHS // 4


def _deg_body(dstp_ref, degp_ref, didx, ones_rows, zbuf, acc,
              s0, s1, s2, s3, isem):
    ssem = [s0, s1, s2, s3]
    c = lax.axis_index("c")
    s = lax.axis_index("s")
    w = c * NS + s
    pltpu.async_copy(dstp_ref.at[pl.ds(w * DEG_NCHS, DEG_NCHS)], didx, isem)
    ids16 = lax.iota(jnp.int32, 16)
    e0vec = jnp.where(ids16 == 0, 1.0, 0.0).astype(jnp.float32)
    zeros16 = jnp.zeros((16,), jnp.float32)
    nrow = NPAD // NS

    def fill(i, carry):
        zbuf[i] = zeros16

        @pl.when(i < CH)
        def _():
            ones_rows[i] = e0vec

        return carry

    lax.fori_loop(0, nrow, fill, 0)
    pltpu.sync_copy(zbuf, acc.at[pl.ds(s * nrow, nrow)])
    pltpu.make_async_copy(dstp_ref.at[pl.ds(w * DEG_NCHS, DEG_NCHS)], didx,
                          isem).wait()
    plsc.subcore_barrier()

    def s_start(j, b):
        pltpu.async_copy(ones_rows, acc.at[didx.at[j]], ssem[b], add=True)

    def s_wait(j, b):
        pltpu.make_async_copy(ones_rows, acc.at[didx.at[j]], ssem[b]).wait()

    def blk(k, carry):
        for b in range(4):
            j = 4 * k + b

            @pl.when(k > 0)
            def _():
                s_wait(j - 4, b)

            s_start(j, b)
        return carry

    lax.fori_loop(0, DEG_NBLK, blk, 0)
    for b in range(4):
        s_wait(DEG_NCHS - 4 + b, b)
    plsc.subcore_barrier()
    pltpu.sync_copy(acc.at[pl.ds(s * nrow, nrow)],
                    degp_ref.at[c, pl.ds(s * nrow, nrow)])


_deg_call = pl.kernel(
    _deg_body,
    out_type=jax.ShapeDtypeStruct((NC, NPAD, 16), jnp.float32),
    mesh=_MESH,
    scratch_types=[
        pltpu.VMEM((DEG_NCHS, CH), jnp.int32),
        pltpu.VMEM((CH, 16), jnp.float32),
        pltpu.VMEM((NPAD // NS, 16), jnp.float32),
        pltpu.VMEM_SHARED((NPAD, 16), jnp.float32),
        pltpu.SemaphoreType.DMA,
        pltpu.SemaphoreType.DMA,
        pltpu.SemaphoreType.DMA,
        pltpu.SemaphoreType.DMA,
        pltpu.SemaphoreType.DMA,
    ],
    compiler_params=_SC_PARAMS,
)

_R = 1000
_NR = N // _R
_PREC = lax.Precision.DEFAULT


def _dinv(deg_ref):
    p = deg_ref[0, :, 0] + deg_ref[1, :, 0] + 1.0
    return lax.rsqrt(p)[:, None]


def _mm1_body(x_ref, w_ref, deg_ref, o_ref):
    dinv = _dinv(deg_ref)
    o_ref[0] = jnp.dot(x_ref[...], w_ref[0], precision=_PREC,
                       preferred_element_type=jnp.float32) * dinv


def _mid_body(m0_ref, m1_ref, deg_ref, b1_ref, w_ref, o_ref):
    dinv = _dinv(deg_ref)
    h = jnp.concatenate([m0_ref[0], m1_ref[0]], axis=1)
    x2 = jnp.maximum(h * dinv + b1_ref[...], 0.0)
    o_ref[0] = jnp.dot(x2, w_ref[0], precision=_PREC,
                       preferred_element_type=jnp.float32) * dinv


def _fin_body(m0_ref, m1_ref, deg_ref, b2_ref, o_ref):
    dinv = _dinv(deg_ref)
    h = jnp.concatenate([m0_ref[0], m1_ref[0]], axis=1)
    o_ref[...] = h * dinv + b2_ref[...]


def _mm1(x, W1, deg2d):
    return pl.pallas_call(
        _mm1_body,
        grid=(NC, _NR),
        in_specs=[
            pl.BlockSpec((_R, D), lambda c, i: (i, 0)),
            pl.BlockSpec((1, D, HD), lambda c, i: (c, 0, 0)),
            pl.BlockSpec((NC, _R, 16), lambda c, i: (0, i, 0)),
        ],
        out_specs=pl.BlockSpec((1, _R, HD), lambda c, i: (c, i, 0)),
        out_shape=jax.ShapeDtypeStruct((NC, N, HD), jnp.float32),
    )(x, W1, deg2d)


def _mid(m, deg2d, b1r, W2):
    return pl.pallas_call(
        _mid_body,
        grid=(NC, _NR),
        in_specs=[
            pl.BlockSpec((1, _R, HD), lambda c, i: (0, i, 0)),
            pl.BlockSpec((1, _R, HD), lambda c, i: (1, i, 0)),
            pl.BlockSpec((NC, _R, 16), lambda c, i: (0, i, 0)),
            pl.BlockSpec((1, D), lambda c, i: (0, 0)),
            pl.BlockSpec((1, D, HD), lambda c, i: (c, 0, 0)),
        ],
        out_specs=pl.BlockSpec((1, _R, HD), lambda c, i: (c, i, 0)),
        out_shape=jax.ShapeDtypeStruct((NC, N, HD), jnp.float32),
    )(m, m, deg2d, b1r, W2)


def _fin(m, deg2d, b2r):
    return pl.pallas_call(
        _fin_body,
        grid=(_NR,),
        in_specs=[
            pl.BlockSpec((1, _R, HD), lambda i: (0, i, 0)),
            pl.BlockSpec((1, _R, HD), lambda i: (1, i, 0)),
            pl.BlockSpec((NC, _R, 16), lambda i: (0, i, 0)),
            pl.BlockSpec((1, D), lambda i: (0, 0)),
        ],
        out_specs=pl.BlockSpec((_R, D), lambda i: (i, 0)),
        out_shape=jax.ShapeDtypeStruct((N, D), jnp.float32),
    )(m, m, deg2d, b2r)


def kernel(x, edge_index, W1, b1, W2, b2):
    src = edge_index[0]
    dst = edge_index[1]
    npad = EPAD - E
    srcs = jnp.concatenate([src, jnp.zeros((npad,), jnp.int32)]).reshape(
        EPAD // CH, CH)
    dstp = jnp.concatenate([dst, jnp.full((npad,), N, jnp.int32)])
    dstp = dstp.reshape(EPAD // CH, CH)
    deg2d = _deg_call(dstp)
    W1s = jnp.stack([W1[:, :HD], W1[:, HD:]])
    W2s = jnp.stack([W2[:, :HD], W2[:, HD:]])
    g1 = _mm1(x, W1s, deg2d)
    m1 = _mp_call(g1, srcs, dstp)
    g2 = _mid(m1, deg2d, b1[None], W2s)
    m2 = _mp_call(g2, srcs, dstp)
    return _fin(m2, deg2d, b2[None])


# speedup vs baseline: 13.5766x; 1.0333x over previous
import jax
import jax.numpy as jnp
from jax import lax
from jax.experimental import pallas as pl
from jax.experimental.pallas import tpu as pltpu
from jax.experimental.pallas import tpu_sc as plsc

N = 10000
E = 320000
D = 128
HD = 64
NC, NS = 2, 16
RPS = N // NS
CH = 128
NCHUNK = E // CH
BASE_CH = NCHUNK // NS
EXTRA_CH = NCHUNK - BASE_CH * NS
EPS = E // (NC * NS)
STG = 2000
NPAD = 10240

_MESH = plsc.VectorSubcoreMesh(
    core_axis_name="c", subcore_axis_name="s", num_cores=NC, num_subcores=NS)
_SC_PARAMS = pltpu.CompilerParams(use_tc_tiling_on_sc=False)


EPAD = 327680
NCHS = EPAD // (NS * CH)
RING = 8
HR = RING // 2
PASS_CH = NCHS // 2
NBLK = PASS_CH // RING
AROW = N + 48


def _mp_body(g_ref, srcs_ref, dstp_ref, out_ref, sidx, didx,
             rows0, rows1, rows2, rows3, rows4, rows5, rows6, rows7, acc,
             g0, g1, g2, g3, g4, g5, g6, g7,
             s0, s1, s2, s3, s4, s5, s6, s7, isem):
    rows = [rows0, rows1, rows2, rows3, rows4, rows5, rows6, rows7]
    gsem = [g0, g1, g2, g3, g4, g5, g6, g7]
    ssem = [s0, s1, s2, s3, s4, s5, s6, s7]
    c = lax.axis_index("c")
    s = lax.axis_index("s")
    r0 = s * RPS

    def stage_idx(p):
        off = s * NCHS + p * PASS_CH
        pltpu.async_copy(srcs_ref.at[pl.ds(off, PASS_CH)], sidx, isem)
        pltpu.async_copy(dstp_ref.at[pl.ds(off, PASS_CH)], didx, isem)

    def wait_idx(p):
        off = s * NCHS + p * PASS_CH
        pltpu.make_async_copy(
            srcs_ref.at[pl.ds(off, PASS_CH)], sidx, isem).wait()
        pltpu.make_async_copy(
            dstp_ref.at[pl.ds(off, PASS_CH)], didx, isem).wait()

    stage_idx(0)
    pltpu.sync_copy(g_ref.at[c, pl.ds(r0, RPS)], acc.at[pl.ds(r0, RPS)])
    wait_idx(0)
    plsc.subcore_barrier()

    def g_start(j, b):
        pltpu.async_copy(g_ref.at[c].at[sidx.at[j]], rows[b], gsem[b])

    def g_wait(j, b):
        pltpu.make_async_copy(g_ref.at[c].at[sidx.at[j]], rows[b], gsem[b]).wait()

    def s_start(j, b):
        pltpu.async_copy(rows[b], acc.at[didx.at[j]], ssem[b], add=True)

    def s_wait(j, b):
        pltpu.make_async_copy(rows[b], acc.at[didx.at[j]], ssem[b]).wait()

    def blk(k, carry):
        for b in range(RING):
            j = RING * k + b
            nb = (b + HR) % RING
            g_wait(j, b)
            s_start(j, b)
            if b < HR:
                @pl.when(k > 0)
                def _():
                    s_wait(j - HR, nb)
                g_start(j + HR, nb)
            else:
                s_wait(j - HR, nb)

                @pl.when(k + 1 < NBLK)
                def _():
                    g_start(j + HR, nb)
        return carry

    for p in range(2):
        if p > 0:
            stage_idx(p)
            wait_idx(p)
        for b in range(HR):
            g_start(b, b)
        lax.fori_loop(0, NBLK, blk, 0)
        for b in range(HR):
            s_wait(PASS_CH - HR + b, HR + b)
    plsc.subcore_barrier()
    pltpu.sync_copy(acc.at[pl.ds(r0, RPS)], out_ref.at[c, pl.ds(r0, RPS)])


_mp_call = pl.kernel(
    _mp_body,
    out_type=jax.ShapeDtypeStruct((NC, N, HD), jnp.float32),
    mesh=_MESH,
    scratch_types=(
        [
            pltpu.VMEM((PASS_CH, CH), jnp.int32),
            pltpu.VMEM((PASS_CH, CH), jnp.int32),
        ]
        + [pltpu.VMEM((CH, HD), jnp.float32)] * RING
        + [pltpu.VMEM_SHARED((AROW, HD), jnp.float32)]
        + [pltpu.SemaphoreType.DMA] * RING
        + [pltpu.SemaphoreType.DMA] * RING
        + [pltpu.SemaphoreType.DMA]
    ),
    compiler_params=_SC_PARAMS,
)


DEG_NCHS = EPAD // (NC * NS * CH)
DEG_NBLK = DEG_NCHS // 4


def _deg_body(dstp_ref, degp_ref, didx, ones_rows, zbuf, acc,
              s0, s1, s2, s3, isem):
    ssem = [s0, s1, s2, s3]
    c = lax.axis_index("c")
    s = lax.axis_index("s")
    w = c * NS + s
    pltpu.async_copy(dstp_ref.at[pl.ds(w * DEG_NCHS, DEG_NCHS)], didx, isem)
    ids16 = lax.iota(jnp.int32, 16)
    e0vec = jnp.where(ids16 == 0, 1.0, 0.0).astype(jnp.float32)
    zeros16 = jnp.zeros((16,), jnp.float32)
    nrow = NPAD // NS

    def fill(i, carry):
        zbuf[i] = zeros16

        @pl.when(i < CH)
        def _():
            ones_rows[i] = e0vec

        return carry

    lax.fori_loop(0, nrow, fill, 0)
    pltpu.sync_copy(zbuf, acc.at[pl.ds(s * nrow, nrow)])
    pltpu.make_async_copy(dstp_ref.at[pl.ds(w * DEG_NCHS, DEG_NCHS)], didx,
                          isem).wait()
    plsc.subcore_barrier()

    def s_start(j, b):
        pltpu.async_copy(ones_rows, acc.at[didx.at[j]], ssem[b], add=True)

    def s_wait(j, b):
        pltpu.make_async_copy(ones_rows, acc.at[didx.at[j]], ssem[b]).wait()

    def blk(k, carry):
        for b in range(4):
            j = 4 * k + b

            @pl.when(k > 0)
            def _():
                s_wait(j - 4, b)

            s_start(j, b)
        return carry

    lax.fori_loop(0, DEG_NBLK, blk, 0)
    for b in range(4):
        s_wait(DEG_NCHS - 4 + b, b)
    plsc.subcore_barrier()
    pltpu.sync_copy(acc.at[pl.ds(s * nrow, nrow)],
                    degp_ref.at[c, pl.ds(s * nrow, nrow)])


_deg_call = pl.kernel(
    _deg_body,
    out_type=jax.ShapeDtypeStruct((NC, NPAD, 16), jnp.float32),
    mesh=_MESH,
    scratch_types=[
        pltpu.VMEM((DEG_NCHS, CH), jnp.int32),
        pltpu.VMEM((CH, 16), jnp.float32),
        pltpu.VMEM((NPAD // NS, 16), jnp.float32),
        pltpu.VMEM_SHARED((NPAD, 16), jnp.float32),
        pltpu.SemaphoreType.DMA,
        pltpu.SemaphoreType.DMA,
        pltpu.SemaphoreType.DMA,
        pltpu.SemaphoreType.DMA,
        pltpu.SemaphoreType.DMA,
    ],
    compiler_params=_SC_PARAMS,
)

_R = 1000
_NR = N // _R
_PREC = lax.Precision.DEFAULT


def _dinv(deg_ref):
    p = deg_ref[0, :, 0] + deg_ref[1, :, 0] + 1.0
    return lax.rsqrt(p)[:, None]


def _mm1_body(x_ref, w_ref, deg_ref, o_ref):
    dinv = _dinv(deg_ref)
    o_ref[0] = jnp.dot(x_ref[...], w_ref[0], precision=_PREC,
                       preferred_element_type=jnp.float32) * dinv


def _mid_body(m0_ref, m1_ref, deg_ref, b1_ref, w_ref, o_ref):
    dinv = _dinv(deg_ref)
    h = jnp.concatenate([m0_ref[0], m1_ref[0]], axis=1)
    x2 = jnp.maximum(h * dinv + b1_ref[...], 0.0)
    o_ref[0] = jnp.dot(x2, w_ref[0], precision=_PREC,
                       preferred_element_type=jnp.float32) * dinv


def _fin_body(m0_ref, m1_ref, deg_ref, b2_ref, o_ref):
    dinv = _dinv(deg_ref)
    h = jnp.concatenate([m0_ref[0], m1_ref[0]], axis=1)
    o_ref[...] = h * dinv + b2_ref[...]


def _mm1(x, W1, deg2d):
    return pl.pallas_call(
        _mm1_body,
        grid=(NC, _NR),
        in_specs=[
            pl.BlockSpec((_R, D), lambda c, i: (i, 0)),
            pl.BlockSpec((1, D, HD), lambda c, i: (c, 0, 0)),
            pl.BlockSpec((NC, _R, 16), lambda c, i: (0, i, 0)),
        ],
        out_specs=pl.BlockSpec((1, _R, HD), lambda c, i: (c, i, 0)),
        out_shape=jax.ShapeDtypeStruct((NC, N, HD), jnp.float32),
    )(x, W1, deg2d)


def _mid(m, deg2d, b1r, W2):
    return pl.pallas_call(
        _mid_body,
        grid=(NC, _NR),
        in_specs=[
            pl.BlockSpec((1, _R, HD), lambda c, i: (0, i, 0)),
            pl.BlockSpec((1, _R, HD), lambda c, i: (1, i, 0)),
            pl.BlockSpec((NC, _R, 16), lambda c, i: (0, i, 0)),
            pl.BlockSpec((1, D), lambda c, i: (0, 0)),
            pl.BlockSpec((1, D, HD), lambda c, i: (c, 0, 0)),
        ],
        out_specs=pl.BlockSpec((1, _R, HD), lambda c, i: (c, i, 0)),
        out_shape=jax.ShapeDtypeStruct((NC, N, HD), jnp.float32),
    )(m, m, deg2d, b1r, W2)


def _fin(m, deg2d, b2r):
    return pl.pallas_call(
        _fin_body,
        grid=(_NR,),
        in_specs=[
            pl.BlockSpec((1, _R, HD), lambda i: (0, i, 0)),
            pl.BlockSpec((1, _R, HD), lambda i: (1, i, 0)),
            pl.BlockSpec((NC, _R, 16), lambda i: (0, i, 0)),
            pl.BlockSpec((1, D), lambda i: (0, 0)),
        ],
        out_specs=pl.BlockSpec((_R, D), lambda i: (i, 0)),
        out_shape=jax.ShapeDtypeStruct((N, D), jnp.float32),
    )(m, m, deg2d, b2r)


def kernel(x, edge_index, W1, b1, W2, b2):
    src = edge_index[0]
    dst = edge_index[1]
    npad = EPAD - E
    srcs = jnp.concatenate([src, jnp.zeros((npad,), jnp.int32)]).reshape(
        EPAD // CH, CH)
    dstp = jnp.concatenate([dst, jnp.full((npad,), N, jnp.int32)])
    dstp = dstp.reshape(EPAD // CH, CH)
    deg2d = _deg_call(dstp)
    W1s = jnp.stack([W1[:, :HD], W1[:, HD:]])
    W2s = jnp.stack([W2[:, :HD], W2[:, HD:]])
    g1 = _mm1(x, W1s, deg2d)
    m1 = _mp_call(g1, srcs, dstp)
    g2 = _mid(m1, deg2d, b1[None], W2s)
    m2 = _mp_call(g2, srcs, dstp)
    return _fin(m2, deg2d, b2[None])
